# baseline (speedup 1.0000x reference)
import functools

import jax
import jax.numpy as jnp
from jax import lax
from jax.experimental import pallas as pl
from jax.experimental.pallas import tpu as pltpu
from jax.experimental.pallas import tpu_sc as plsc

NC = 2
NS = 16
NW = NC * NS
LANES = 16


def _emb_kernel(B, L, E, CROWS):
    CH = CROWS * L
    per_w = (B * L) // NW
    n_chunks = per_w // CH
    mesh = plsc.VectorSubcoreMesh(core_axis_name="c", subcore_axis_name="s")

    @functools.partial(
        pl.kernel,
        mesh=mesh,
        out_type=jax.ShapeDtypeStruct((B * L, E), jnp.float32),
        scratch_types=[
            pltpu.VMEM((CH,), jnp.int32),
            pltpu.VMEM((CH, E), jnp.float32),
            pltpu.VMEM((CH, E), jnp.float32),
            pltpu.SemaphoreType.DMA,
        ],
        compiler_params=pltpu.CompilerParams(use_tc_tiling_on_sc=False),
    )
    def k(src_hbm, wtab_hbm, ptab_hbm, out_hbm, idx_v, rows_v, pat_v, sem):
        wid = lax.axis_index("s") * NC + lax.axis_index("c")
        base = wid * per_w

        for r in range(CROWS):
            pltpu.sync_copy(ptab_hbm.at[pl.ds(0, L)],
                            pat_v.at[pl.ds(r * L, L)])

        def chunk_body(j, carry):
            start = base + j * CH
            pltpu.sync_copy(src_hbm.at[pl.ds(start, CH)], idx_v)
            pltpu.async_copy(wtab_hbm.at[idx_v], rows_v, sem).wait()

            def add_row(r, c2):
                for c in range(E // LANES):
                    sl = pl.ds(c * LANES, LANES)
                    rows_v[r, sl] = rows_v[r, sl] + pat_v[r, sl]
                return c2

            lax.fori_loop(0, CH, add_row, 0)
            pltpu.sync_copy(rows_v, out_hbm.at[pl.ds(start, CH)])
            return carry

        lax.fori_loop(0, n_chunks, chunk_body, 0)

    return k


def kernel(src, seg, word_table, pos_table):
    B, L = src.shape
    V, E = word_table.shape
    src_flat = src.reshape(B * L).astype(jnp.int32)
    k = _emb_kernel(B, L, E, CROWS=4)
    out = k(src_flat, word_table, pos_table)
    return out.reshape(B, L, E)
